# Initial kernel scaffold; baseline (speedup 1.0000x reference)
import jax
import jax.numpy as jnp
from jax.experimental import pallas as pl


def kernel(node_features, edge_index, node_positions, node_radii, W1, b1, W2, b2, W3, b3):
    raise NotImplementedError("write your pallas kernel here")


# speedup vs baseline: 6.0300x; 6.0300x over previous
import functools

import jax
import jax.numpy as jnp
from jax import lax
from jax.experimental import pallas as pl
from jax.experimental.pallas import tpu as pltpu
from jax.experimental.pallas import tpu_sc as plsc

D = 128
N = 10000
E = 320000
H1 = 64
H2 = 32

NC = 2
NS = 16
NW = NC * NS
EW = E // NW
CHUNK = 80
NCHUNK = EW // CHUNK
NPAD = 10240
SL = NPAD // NS


def _proj_body(nf_ref, w1a_ref, w1b_ref, p_ref, q_ref):
    x = nf_ref[...]
    p_ref[...] = jnp.dot(x, w1a_ref[...], preferred_element_type=jnp.float32)
    q_ref[...] = jnp.dot(x, w1b_ref[...], preferred_element_type=jnp.float32)


def _project(nf, w1a, w1b):
    blk = 1000
    grid = N // blk
    return pl.pallas_call(
        _proj_body,
        grid=(grid,),
        in_specs=[
            pl.BlockSpec((blk, D), lambda i: (i, 0)),
            pl.BlockSpec((D, H1), lambda i: (0, 0)),
            pl.BlockSpec((D, H1), lambda i: (0, 0)),
        ],
        out_specs=[
            pl.BlockSpec((blk, H1), lambda i: (i, 0)),
            pl.BlockSpec((blk, H1), lambda i: (i, 0)),
        ],
        out_shape=[
            jax.ShapeDtypeStruct((N, H1), jnp.float32),
            jax.ShapeDtypeStruct((N, H1), jnp.float32),
        ],
    )(nf, w1a, w1b)


def _gather_sum_body(p_hbm, q_hbm, src_hbm, tgt_hbm, out_hbm,
                     idx_s, idx_t, pa, qa, pb, qb, sem_a, sem_b):
    wid = lax.axis_index("s") * NC + lax.axis_index("c")
    base = wid * EW

    pltpu.sync_copy(src_hbm.at[wid], idx_s)
    pltpu.sync_copy(tgt_hbm.at[wid], idx_t)

    def issue(j, prow, qrow, sem):
        cp = pltpu.async_copy(p_hbm.at[idx_s.at[j]], prow, sem)
        cq = pltpu.async_copy(q_hbm.at[idx_t.at[j]], qrow, sem)
        return cp, cq

    def drain(j, prow, qrow, sem):
        pltpu.make_async_copy(p_hbm.at[idx_s.at[j]], prow, sem).wait()
        pltpu.make_async_copy(q_hbm.at[idx_t.at[j]], qrow, sem).wait()

        def add_row(r, _):
            for k in range(H1 // 16):
                sl = pl.ds(k * 16, 16)
                prow[r, sl] += qrow[r, sl]
            return 0

        lax.fori_loop(0, CHUNK, add_row, 0)
        pltpu.sync_copy(prow, out_hbm.at[pl.ds(base + j * CHUNK, CHUNK)])

    npair = NCHUNK // 2
    issue(0, pa, qa, sem_a)

    def pair_body(g, _):
        j0 = g * 2
        issue(j0 + 1, pb, qb, sem_b)
        drain(j0, pa, qa, sem_a)

        @pl.when(j0 + 2 < NCHUNK)
        def _():
            issue(j0 + 2, pa, qa, sem_a)

        drain(j0 + 1, pb, qb, sem_b)
        return 0

    lax.fori_loop(0, npair, pair_body, 0)
    drain(NCHUNK - 1, pa, qa, sem_a)


def _gather_sum(p, q, src3, tgt3):
    mesh = plsc.VectorSubcoreMesh(core_axis_name="c", subcore_axis_name="s")
    kern = pl.kernel(
        _gather_sum_body,
        out_type=jax.ShapeDtypeStruct((E, H1), jnp.float32),
        mesh=mesh,
        scratch_types=[
            pltpu.VMEM((NCHUNK, CHUNK), jnp.int32),
            pltpu.VMEM((NCHUNK, CHUNK), jnp.int32),
            pltpu.VMEM((CHUNK, H1), jnp.float32),
            pltpu.VMEM((CHUNK, H1), jnp.float32),
            pltpu.VMEM((CHUNK, H1), jnp.float32),
            pltpu.VMEM((CHUNK, H1), jnp.float32),
            pltpu.SemaphoreType.DMA,
            pltpu.SemaphoreType.DMA,
        ],
        compiler_params=pltpu.CompilerParams(use_tc_tiling_on_sc=False, needs_layout_passes=False),
    )
    return kern(p, q, src3, tgt3)


def _tail_body(s_ref, b1_ref, w2_ref, b2_ref, w3_ref, b3_ref,
               score_ref, coeff_ref, ssum_ref, viol_ref):
    i = pl.program_id(0)
    h = jnp.maximum(s_ref[...] + b1_ref[...], 0.0)
    h2 = jnp.dot(h, w2_ref[...], preferred_element_type=jnp.float32)
    h2 = jnp.maximum(h2 + b2_ref[...], 0.0)
    z = jnp.dot(h2, w3_ref[...], preferred_element_type=jnp.float32) + b3_ref[...]
    score = jax.nn.sigmoid(z)
    mask = (score < 0.7).astype(jnp.float32)
    score_ref[...] = score
    coeff_ref[...] = 0.05 * mask * (1.0 - score)

    @pl.when(i == 0)
    def _():
        ssum_ref[...] = jnp.zeros_like(ssum_ref)
        viol_ref[...] = jnp.zeros_like(viol_ref)

    ssum_ref[...] = ssum_ref[...] + jnp.sum(score).reshape(1, 1)
    viol_ref[...] = viol_ref[...] + jnp.sum(mask).reshape(1, 1)


def _mlp_tail(s, b1, w2, b2, w3, b3):
    blk = 2000
    grid = E // blk
    return pl.pallas_call(
        _tail_body,
        grid=(grid,),
        in_specs=[
            pl.BlockSpec((blk, H1), lambda i: (i, 0)),
            pl.BlockSpec((1, H1), lambda i: (0, 0)),
            pl.BlockSpec((H1, H2), lambda i: (0, 0)),
            pl.BlockSpec((1, H2), lambda i: (0, 0)),
            pl.BlockSpec((H2, 1), lambda i: (0, 0)),
            pl.BlockSpec((1, 1), lambda i: (0, 0)),
        ],
        out_specs=[
            pl.BlockSpec((blk, 1), lambda i: (i, 0)),
            pl.BlockSpec((blk, 1), lambda i: (i, 0)),
            pl.BlockSpec((1, 1), lambda i: (0, 0)),
            pl.BlockSpec((1, 1), lambda i: (0, 0)),
        ],
        out_shape=[
            jax.ShapeDtypeStruct((E, 1), jnp.float32),
            jax.ShapeDtypeStruct((E, 1), jnp.float32),
            jax.ShapeDtypeStruct((1, 1), jnp.float32),
            jax.ShapeDtypeStruct((1, 1), jnp.float32),
        ],
    )(s, b1.reshape(1, H1), w2, b2.reshape(1, H2), w3, b3.reshape(1, 1))


def _segsum_body(coeff_hbm, src_hbm, tgt_hbm, out_hbm,
                 coeff_v, src_v, tgt_v, w_loc, acc_v, tmp_v, shared):
    cid = lax.axis_index("c")
    sid = lax.axis_index("s")
    wid = sid * NC + cid
    base = wid * EW

    pltpu.sync_copy(coeff_hbm.at[pl.ds(base, EW)], coeff_v)
    pltpu.sync_copy(src_hbm.at[pl.ds(base, EW)], src_v)
    pltpu.sync_copy(tgt_hbm.at[pl.ds(base, EW)], tgt_v)

    def zero_body(i, _):
        w_loc[pl.ds(i * 16, 16)] = jnp.zeros((16,), jnp.float32)
        return 0

    lax.fori_loop(0, NPAD // 16, zero_body, 0)

    def edge_body(i, _):
        sl = pl.ds(i * 16, 16)
        c = coeff_v[sl]
        plsc.addupdate_scatter(w_loc, [src_v[sl]], c)
        plsc.addupdate_scatter(w_loc, [tgt_v[sl]], c)
        return 0

    lax.fori_loop(0, EW // 16, edge_body, 0)

    pltpu.sync_copy(w_loc, shared.at[sid])
    plsc.subcore_barrier()

    col = pl.ds(sid * SL, SL)
    pltpu.sync_copy(shared.at[0, col], acc_v)
    for r in range(1, NS):
        pltpu.sync_copy(shared.at[r, col], tmp_v)

        def add_body(i, _):
            sl = pl.ds(i * 16, 16)
            acc_v[sl] += tmp_v[sl]
            return 0

        lax.fori_loop(0, SL // 16, add_body, 0)

    pltpu.sync_copy(acc_v, out_hbm.at[cid, col])


def _segment_sum(coeff, src, tgt):
    mesh = plsc.VectorSubcoreMesh(core_axis_name="c", subcore_axis_name="s")
    kern = pl.kernel(
        _segsum_body,
        out_type=jax.ShapeDtypeStruct((NC, NPAD), jnp.float32),
        mesh=mesh,
        scratch_types=[
            pltpu.VMEM((EW,), jnp.float32),
            pltpu.VMEM((EW,), jnp.int32),
            pltpu.VMEM((EW,), jnp.int32),
            pltpu.VMEM((NPAD,), jnp.float32),
            pltpu.VMEM((SL,), jnp.float32),
            pltpu.VMEM((SL,), jnp.float32),
            pltpu.VMEM_SHARED((NS, NPAD), jnp.float32),
        ],
        compiler_params=pltpu.CompilerParams(use_tc_tiling_on_sc=False, needs_layout_passes=False),
    )
    return kern(coeff, src, tgt)


def _update_body(nf_ref, w_ref, out_ref):
    x = nf_ref[...]
    out_ref[...] = x + w_ref[...] * jnp.tanh(x)


def _apply_update(nf, w_col):
    blk = 1000
    grid = N // blk
    return pl.pallas_call(
        _update_body,
        grid=(grid,),
        in_specs=[
            pl.BlockSpec((blk, D), lambda i: (i, 0)),
            pl.BlockSpec((blk, 1), lambda i: (i, 0)),
        ],
        out_specs=pl.BlockSpec((blk, D), lambda i: (i, 0)),
        out_shape=jax.ShapeDtypeStruct((N, D), jnp.float32),
    )(nf, w_col)


@jax.jit
def _run(node_features, edge_index, W1, b1, W2, b2, W3, b3):
    src = edge_index[0].astype(jnp.int32)
    tgt = edge_index[1].astype(jnp.int32)
    src3 = src.reshape(NW, NCHUNK, CHUNK)
    tgt3 = tgt.reshape(NW, NCHUNK, CHUNK)

    p, q = _project(node_features, W1[:D], W1[D:])
    s = _gather_sum(p, q, src3, tgt3)
    score, coeff, ssum, viol = _mlp_tail(s, b1, W2, b2, W3, b3)
    wpair = _segment_sum(coeff.reshape(E), src, tgt)
    w_col = (wpair[0, :N] + wpair[1, :N]).reshape(N, 1)
    updated = _apply_update(node_features, w_col)

    scores = score.reshape(E)
    avg = ssum[0, 0] / jnp.float32(E)
    nviol = viol[0, 0].astype(jnp.int32)
    return updated, scores, avg, nviol


def kernel(node_features, edge_index, node_positions, node_radii,
           W1, b1, W2, b2, W3, b3):
    return _run(node_features, edge_index, W1, b1, W2, b2, W3, b3)


# speedup vs baseline: 6.8059x; 1.1287x over previous
import functools

import jax
import jax.numpy as jnp
from jax import lax
from jax.experimental import pallas as pl
from jax.experimental.pallas import tpu as pltpu
from jax.experimental.pallas import tpu_sc as plsc

D = 128
N = 10000
E = 320000
H1 = 64
H2 = 32

NC = 2
NS = 16
NW = NC * NS
EW = E // NW
CHUNK = 80
NCHUNK = EW // CHUNK
NPAD = 10240
SL = NPAD // NS


def _proj_body(nf_ref, w1a_ref, w1b_ref, p_ref, q_ref):
    x = nf_ref[...]
    p_ref[...] = jnp.dot(x, w1a_ref[...], preferred_element_type=jnp.float32)
    q_ref[...] = jnp.dot(x, w1b_ref[...], preferred_element_type=jnp.float32)


def _project(nf, w1a, w1b):
    blk = 1000
    grid = N // blk
    return pl.pallas_call(
        _proj_body,
        grid=(grid,),
        in_specs=[
            pl.BlockSpec((blk, D), lambda i: (i, 0)),
            pl.BlockSpec((D, H1), lambda i: (0, 0)),
            pl.BlockSpec((D, H1), lambda i: (0, 0)),
        ],
        out_specs=[
            pl.BlockSpec((blk, H1), lambda i: (i, 0)),
            pl.BlockSpec((blk, H1), lambda i: (i, 0)),
        ],
        out_shape=[
            jax.ShapeDtypeStruct((N, H1), jnp.float32),
            jax.ShapeDtypeStruct((N, H1), jnp.float32),
        ],
    )(nf, w1a, w1b)


def _gather_sum_body(p_hbm, q_hbm, src_hbm, tgt_hbm, out_hbm,
                     idx_s, idx_t, pa, qa, pb, qb, sem_a, sem_b):
    wid = lax.axis_index("s") * NC + lax.axis_index("c")
    base = wid * EW

    pltpu.sync_copy(src_hbm.at[wid], idx_s)
    pltpu.sync_copy(tgt_hbm.at[wid], idx_t)

    def issue(j, prow, qrow, sem):
        cp = pltpu.async_copy(p_hbm.at[idx_s.at[j]], prow, sem)
        cq = pltpu.async_copy(q_hbm.at[idx_t.at[j]], qrow, sem)
        return cp, cq

    def drain(j, prow, qrow, sem):
        pltpu.make_async_copy(p_hbm.at[idx_s.at[j]], prow, sem).wait()
        pltpu.make_async_copy(q_hbm.at[idx_t.at[j]], qrow, sem).wait()

        def add_row(r, _):
            for k in range(H1 // 16):
                sl = pl.ds(k * 16, 16)
                prow[r, sl] += qrow[r, sl]
            return 0

        lax.fori_loop(0, CHUNK, add_row, 0)
        pltpu.sync_copy(prow, out_hbm.at[pl.ds(base + j * CHUNK, CHUNK)])

    npair = NCHUNK // 2
    issue(0, pa, qa, sem_a)

    def pair_body(g, _):
        j0 = g * 2
        issue(j0 + 1, pb, qb, sem_b)
        drain(j0, pa, qa, sem_a)

        @pl.when(j0 + 2 < NCHUNK)
        def _():
            issue(j0 + 2, pa, qa, sem_a)

        drain(j0 + 1, pb, qb, sem_b)
        return 0

    lax.fori_loop(0, npair, pair_body, 0)
    drain(NCHUNK - 1, pa, qa, sem_a)


def _gather_sum(p, q, src3, tgt3):
    mesh = plsc.VectorSubcoreMesh(core_axis_name="c", subcore_axis_name="s")
    kern = pl.kernel(
        _gather_sum_body,
        out_type=jax.ShapeDtypeStruct((E, H1), jnp.float32),
        mesh=mesh,
        scratch_types=[
            pltpu.VMEM((NCHUNK, CHUNK), jnp.int32),
            pltpu.VMEM((NCHUNK, CHUNK), jnp.int32),
            pltpu.VMEM((CHUNK, H1), jnp.float32),
            pltpu.VMEM((CHUNK, H1), jnp.float32),
            pltpu.VMEM((CHUNK, H1), jnp.float32),
            pltpu.VMEM((CHUNK, H1), jnp.float32),
            pltpu.SemaphoreType.DMA,
            pltpu.SemaphoreType.DMA,
        ],
        compiler_params=pltpu.CompilerParams(use_tc_tiling_on_sc=False, needs_layout_passes=False),
    )
    return kern(p, q, src3, tgt3)


def _tail_body(s_ref, b1_ref, w2_ref, b2_ref, w3_ref, z_ref):
    h = jnp.maximum(s_ref[...] + b1_ref[...], 0.0)
    h2 = jnp.dot(h, w2_ref[...], preferred_element_type=jnp.float32)
    h2 = jnp.maximum(h2 + b2_ref[...], 0.0)
    z_ref[...] = jnp.dot(h2, w3_ref[...], preferred_element_type=jnp.float32)


def _mlp_tail(s, b1, w2, b2, w3):
    blk = 2000
    grid = E // blk
    return pl.pallas_call(
        _tail_body,
        grid=(grid,),
        in_specs=[
            pl.BlockSpec((blk, H1), lambda i: (i, 0)),
            pl.BlockSpec((1, H1), lambda i: (0, 0)),
            pl.BlockSpec((H1, H2), lambda i: (0, 0)),
            pl.BlockSpec((1, H2), lambda i: (0, 0)),
            pl.BlockSpec((H2, 1), lambda i: (0, 0)),
        ],
        out_specs=pl.BlockSpec((blk, 1), lambda i: (i, 0)),
        out_shape=jax.ShapeDtypeStruct((E, 1), jnp.float32),
    )(s, b1.reshape(1, H1), w2, b2.reshape(1, H2), w3)


def _segsum_body(z_hbm, src_hbm, tgt_hbm, b3_hbm,
                 score_hbm, w_hbm, ssum_hbm, viol_hbm,
                 z_v, src_v, tgt_v, score_v, b3_v, sacc, vacc,
                 w_loc, acc_v, tmp_v, shared):
    cid = lax.axis_index("c")
    sid = lax.axis_index("s")
    wid = sid * NC + cid
    base = wid * EW

    pltpu.sync_copy(z_hbm.at[pl.ds(base, EW)], z_v)
    pltpu.sync_copy(src_hbm.at[pl.ds(base, EW)], src_v)
    pltpu.sync_copy(tgt_hbm.at[pl.ds(base, EW)], tgt_v)
    pltpu.sync_copy(b3_hbm, b3_v)

    def zero_body(i, _):
        w_loc[pl.ds(i * 16, 16)] = jnp.zeros((16,), jnp.float32)
        return 0

    lax.fori_loop(0, NPAD // 16, zero_body, 0)
    sacc[...] = jnp.zeros((16,), jnp.float32)
    vacc[...] = jnp.zeros((16,), jnp.float32)

    def edge_body(i, _):
        sl = pl.ds(i * 16, 16)
        zb = z_v[sl] + b3_v[...]
        sc = 1.0 / (1.0 + jnp.exp(-zb))
        m = sc < 0.7
        coeff = jnp.where(m, 0.05 * (1.0 - sc), 0.0)
        score_v[sl] = sc
        sacc[...] += sc
        vacc[...] += jnp.where(m, 1.0, 0.0)
        plsc.addupdate_scatter(w_loc, [src_v[sl]], coeff)
        plsc.addupdate_scatter(w_loc, [tgt_v[sl]], coeff)
        return 0

    lax.fori_loop(0, EW // 16, edge_body, 0)

    pltpu.sync_copy(score_v, score_hbm.at[pl.ds(base, EW)])
    pltpu.sync_copy(sacc, ssum_hbm.at[wid])
    pltpu.sync_copy(vacc, viol_hbm.at[wid])

    pltpu.sync_copy(w_loc, shared.at[sid])
    plsc.subcore_barrier()

    col = pl.ds(sid * SL, SL)
    pltpu.sync_copy(shared.at[0, col], acc_v)
    for r in range(1, NS):
        pltpu.sync_copy(shared.at[r, col], tmp_v)

        def add_body(i, _):
            sl = pl.ds(i * 16, 16)
            acc_v[sl] += tmp_v[sl]
            return 0

        lax.fori_loop(0, SL // 16, add_body, 0)

    pltpu.sync_copy(acc_v, w_hbm.at[cid, col])


def _segment_sum(z, src, tgt, b3x16):
    mesh = plsc.VectorSubcoreMesh(core_axis_name="c", subcore_axis_name="s")
    kern = pl.kernel(
        _segsum_body,
        out_type=[
            jax.ShapeDtypeStruct((E,), jnp.float32),
            jax.ShapeDtypeStruct((NC, NPAD), jnp.float32),
            jax.ShapeDtypeStruct((NW, 16), jnp.float32),
            jax.ShapeDtypeStruct((NW, 16), jnp.float32),
        ],
        mesh=mesh,
        scratch_types=[
            pltpu.VMEM((EW,), jnp.float32),
            pltpu.VMEM((EW,), jnp.int32),
            pltpu.VMEM((EW,), jnp.int32),
            pltpu.VMEM((EW,), jnp.float32),
            pltpu.VMEM((16,), jnp.float32),
            pltpu.VMEM((16,), jnp.float32),
            pltpu.VMEM((16,), jnp.float32),
            pltpu.VMEM((NPAD,), jnp.float32),
            pltpu.VMEM((SL,), jnp.float32),
            pltpu.VMEM((SL,), jnp.float32),
            pltpu.VMEM_SHARED((NS, NPAD), jnp.float32),
        ],
        compiler_params=pltpu.CompilerParams(use_tc_tiling_on_sc=False, needs_layout_passes=False),
    )
    return kern(z, src, tgt, b3x16)


def _update_body(nf_ref, w_ref, ss_ref, vv_ref, out_ref, ssum_ref, viol_ref):
    i = pl.program_id(0)
    x = nf_ref[...]
    out_ref[...] = x + w_ref[...] * jnp.tanh(x)

    @pl.when(i == 0)
    def _():
        ssum_ref[...] = jnp.sum(ss_ref[...]).reshape(1, 1)
        viol_ref[...] = jnp.sum(vv_ref[...]).reshape(1, 1)


def _apply_update(nf, w_col, ssum_p, viol_p):
    blk = 1000
    grid = N // blk
    return pl.pallas_call(
        _update_body,
        grid=(grid,),
        in_specs=[
            pl.BlockSpec((blk, D), lambda i: (i, 0)),
            pl.BlockSpec((blk, 1), lambda i: (i, 0)),
            pl.BlockSpec((NW, 16), lambda i: (0, 0)),
            pl.BlockSpec((NW, 16), lambda i: (0, 0)),
        ],
        out_specs=[
            pl.BlockSpec((blk, D), lambda i: (i, 0)),
            pl.BlockSpec((1, 1), lambda i: (0, 0)),
            pl.BlockSpec((1, 1), lambda i: (0, 0)),
        ],
        out_shape=[
            jax.ShapeDtypeStruct((N, D), jnp.float32),
            jax.ShapeDtypeStruct((1, 1), jnp.float32),
            jax.ShapeDtypeStruct((1, 1), jnp.float32),
        ],
    )(nf, w_col, ssum_p, viol_p)


@jax.jit
def _run(node_features, edge_index, W1, b1, W2, b2, W3, b3):
    src = edge_index[0].astype(jnp.int32)
    tgt = edge_index[1].astype(jnp.int32)
    src3 = src.reshape(NW, NCHUNK, CHUNK)
    tgt3 = tgt.reshape(NW, NCHUNK, CHUNK)

    p, q = _project(node_features, W1[:D], W1[D:])
    s = _gather_sum(p, q, src3, tgt3)
    z = _mlp_tail(s, b1, W2, b2, W3)
    b3x16 = jnp.broadcast_to(b3.reshape(1), (16,))
    scores, wpair, ssum_p, viol_p = _segment_sum(z.reshape(E), src, tgt, b3x16)
    w_col = (wpair[0, :N] + wpair[1, :N]).reshape(N, 1)
    updated, ssum, viol = _apply_update(node_features, w_col, ssum_p, viol_p)

    avg = ssum[0, 0] / jnp.float32(E)
    nviol = viol[0, 0].astype(jnp.int32)
    return updated, scores, avg, nviol


def kernel(node_features, edge_index, node_positions, node_radii,
           W1, b1, W2, b2, W3, b3):
    return _run(node_features, edge_index, W1, b1, W2, b2, W3, b3)


# speedup vs baseline: 10.4837x; 1.5404x over previous
import functools

import jax
import jax.numpy as jnp
from jax import lax
from jax.experimental import pallas as pl
from jax.experimental.pallas import tpu as pltpu
from jax.experimental.pallas import tpu_sc as plsc

D = 128
N = 10000
E = 320000
H1 = 64
H2 = 32

NC = 2
NS = 16
NW = NC * NS
EW = E // NW
CHUNK = 80
NCHUNK = EW // CHUNK
NPAD = 10240
SL = NPAD // NS


def _proj_body(nf_ref, w1a_ref, w1b_ref, p_ref, q_ref):
    x = nf_ref[...]
    p_ref[...] = jnp.dot(x, w1a_ref[...], preferred_element_type=jnp.float32)
    q_ref[...] = jnp.dot(x, w1b_ref[...], preferred_element_type=jnp.float32)


def _project(nf, w1a, w1b):
    blk = 1000
    grid = N // blk
    return pl.pallas_call(
        _proj_body,
        grid=(grid,),
        in_specs=[
            pl.BlockSpec((blk, D), lambda i: (i, 0)),
            pl.BlockSpec((D, H1), lambda i: (0, 0)),
            pl.BlockSpec((D, H1), lambda i: (0, 0)),
        ],
        out_specs=[
            pl.BlockSpec((blk, H1), lambda i: (i, 0)),
            pl.BlockSpec((blk, H1), lambda i: (i, 0)),
        ],
        out_shape=[
            jax.ShapeDtypeStruct((N, H1), jnp.float32),
            jax.ShapeDtypeStruct((N, H1), jnp.float32),
        ],
    )(nf, w1a, w1b)


def _gather_sum_body(p_hbm, q_hbm, src_hbm, tgt_hbm, out_hbm,
                     idx_s, idx_t, pa, qa, pb, qb, sem_a, sem_b):
    wid = lax.axis_index("s") * NC + lax.axis_index("c")
    base = wid * EW

    pltpu.sync_copy(src_hbm.at[pl.ds(base, EW)], idx_s)
    pltpu.sync_copy(tgt_hbm.at[pl.ds(base, EW)], idx_t)

    def issue(j, prow, qrow, sem):
        cp = pltpu.async_copy(p_hbm.at[idx_s.at[pl.ds(j * CHUNK, CHUNK)]], prow, sem)
        cq = pltpu.async_copy(q_hbm.at[idx_t.at[pl.ds(j * CHUNK, CHUNK)]], qrow, sem)
        return cp, cq

    def drain(j, prow, qrow, sem):
        pltpu.make_async_copy(p_hbm.at[idx_s.at[pl.ds(j * CHUNK, CHUNK)]], prow, sem).wait()
        pltpu.make_async_copy(q_hbm.at[idx_t.at[pl.ds(j * CHUNK, CHUNK)]], qrow, sem).wait()

        def add_row(r, _):
            for k in range(H1 // 16):
                sl = pl.ds(k * 16, 16)
                prow[r, sl] += qrow[r, sl]
            return 0

        lax.fori_loop(0, CHUNK, add_row, 0)
        pltpu.sync_copy(prow, out_hbm.at[pl.ds(base + j * CHUNK, CHUNK)])

    npair = NCHUNK // 2
    issue(0, pa, qa, sem_a)

    def pair_body(g, _):
        j0 = g * 2
        issue(j0 + 1, pb, qb, sem_b)
        drain(j0, pa, qa, sem_a)

        @pl.when(j0 + 2 < NCHUNK)
        def _():
            issue(j0 + 2, pa, qa, sem_a)

        drain(j0 + 1, pb, qb, sem_b)
        return 0

    lax.fori_loop(0, npair, pair_body, 0)
    drain(NCHUNK - 1, pa, qa, sem_a)


def _gather_sum(p, q, src, tgt):
    mesh = plsc.VectorSubcoreMesh(core_axis_name="c", subcore_axis_name="s")
    kern = pl.kernel(
        _gather_sum_body,
        out_type=jax.ShapeDtypeStruct((E, H1), jnp.float32),
        mesh=mesh,
        scratch_types=[
            pltpu.VMEM((EW,), jnp.int32),
            pltpu.VMEM((EW,), jnp.int32),
            pltpu.VMEM((CHUNK, H1), jnp.float32),
            pltpu.VMEM((CHUNK, H1), jnp.float32),
            pltpu.VMEM((CHUNK, H1), jnp.float32),
            pltpu.VMEM((CHUNK, H1), jnp.float32),
            pltpu.SemaphoreType.DMA,
            pltpu.SemaphoreType.DMA,
        ],
        compiler_params=pltpu.CompilerParams(use_tc_tiling_on_sc=False, needs_layout_passes=False),
    )
    return kern(p, q, src, tgt)


E2 = E // 2


def _tail_body(s_ref, b1_ref, w2_ref, b2_ref, w3_ref, z_ref):
    h = jnp.maximum(s_ref[...] + b1_ref[...], 0.0)
    h2 = jnp.dot(h, w2_ref[...], preferred_element_type=jnp.float32)
    h2 = jnp.maximum(h2 + b2_ref[...], 0.0)
    z_ref[...] = jnp.dot(h2, w3_ref[...], preferred_element_type=jnp.float32)


def _mlp_tail(s2, b1d, w2d, b2d, w3d):
    blk = 2000
    grid = E2 // blk
    return pl.pallas_call(
        _tail_body,
        grid=(grid,),
        in_specs=[
            pl.BlockSpec((blk, 2 * H1), lambda i: (i, 0)),
            pl.BlockSpec((1, 2 * H1), lambda i: (0, 0)),
            pl.BlockSpec((2 * H1, 2 * H2), lambda i: (0, 0)),
            pl.BlockSpec((1, 2 * H2), lambda i: (0, 0)),
            pl.BlockSpec((2 * H2, 2), lambda i: (0, 0)),
        ],
        out_specs=pl.BlockSpec((blk, 2), lambda i: (i, 0)),
        out_shape=jax.ShapeDtypeStruct((E2, 2), jnp.float32),
    )(s2, b1d.reshape(1, 2 * H1), w2d, b2d.reshape(1, 2 * H2), w3d)


def _segsum_body(z_hbm, src_hbm, tgt_hbm, b3_hbm,
                 score_hbm, w_hbm, ssum_hbm, viol_hbm,
                 z_v, src_v, tgt_v, score_v, b3_v, sacc, vacc,
                 w_loc, acc_v, tmp_v, shared):
    cid = lax.axis_index("c")
    sid = lax.axis_index("s")
    wid = sid * NC + cid
    base = wid * EW

    pltpu.sync_copy(z_hbm.at[pl.ds(base, EW)], z_v)
    pltpu.sync_copy(src_hbm.at[pl.ds(base, EW)], src_v)
    pltpu.sync_copy(tgt_hbm.at[pl.ds(base, EW)], tgt_v)
    pltpu.sync_copy(b3_hbm, b3_v)

    def zero_body(i, _):
        w_loc[pl.ds(i * 16, 16)] = jnp.zeros((16,), jnp.float32)
        return 0

    lax.fori_loop(0, NPAD // 16, zero_body, 0)
    sacc[...] = jnp.zeros((16,), jnp.float32)
    vacc[...] = jnp.zeros((16,), jnp.float32)

    def edge_body(i, _):
        sl = pl.ds(i * 16, 16)
        zb = z_v[sl] + b3_v[...]
        sc = 1.0 / (1.0 + jnp.exp(-zb))
        m = sc < 0.7
        coeff = jnp.where(m, 0.05 * (1.0 - sc), 0.0)
        score_v[sl] = sc
        sacc[...] += sc
        vacc[...] += jnp.where(m, 1.0, 0.0)
        plsc.addupdate_scatter(w_loc, [src_v[sl]], coeff)
        plsc.addupdate_scatter(w_loc, [tgt_v[sl]], coeff)
        return 0

    lax.fori_loop(0, EW // 16, edge_body, 0)

    pltpu.sync_copy(score_v, score_hbm.at[pl.ds(base, EW)])
    pltpu.sync_copy(sacc, ssum_hbm.at[wid])
    pltpu.sync_copy(vacc, viol_hbm.at[wid])

    pltpu.sync_copy(w_loc, shared.at[sid])
    plsc.subcore_barrier()

    col = pl.ds(sid * SL, SL)
    pltpu.sync_copy(shared.at[0, col], acc_v)
    for r in range(1, NS):
        pltpu.sync_copy(shared.at[r, col], tmp_v)

        def add_body(i, _):
            sl = pl.ds(i * 16, 16)
            acc_v[sl] += tmp_v[sl]
            return 0

        lax.fori_loop(0, SL // 16, add_body, 0)

    pltpu.sync_copy(acc_v, w_hbm.at[cid, col])


def _segment_sum(z, src, tgt, b3x16):
    mesh = plsc.VectorSubcoreMesh(core_axis_name="c", subcore_axis_name="s")
    kern = pl.kernel(
        _segsum_body,
        out_type=[
            jax.ShapeDtypeStruct((E,), jnp.float32),
            jax.ShapeDtypeStruct((NC, NPAD), jnp.float32),
            jax.ShapeDtypeStruct((NW, 16), jnp.float32),
            jax.ShapeDtypeStruct((NW, 16), jnp.float32),
        ],
        mesh=mesh,
        scratch_types=[
            pltpu.VMEM((EW,), jnp.float32),
            pltpu.VMEM((EW,), jnp.int32),
            pltpu.VMEM((EW,), jnp.int32),
            pltpu.VMEM((EW,), jnp.float32),
            pltpu.VMEM((16,), jnp.float32),
            pltpu.VMEM((16,), jnp.float32),
            pltpu.VMEM((16,), jnp.float32),
            pltpu.VMEM((NPAD,), jnp.float32),
            pltpu.VMEM((SL,), jnp.float32),
            pltpu.VMEM((SL,), jnp.float32),
            pltpu.VMEM_SHARED((NS, NPAD), jnp.float32),
        ],
        compiler_params=pltpu.CompilerParams(use_tc_tiling_on_sc=False, needs_layout_passes=False),
    )
    return kern(z, src, tgt, b3x16)


def _update_body(nf_ref, w_ref, ss_ref, vv_ref, out_ref, ssum_ref, viol_ref):
    i = pl.program_id(0)
    x = nf_ref[...]
    out_ref[...] = x + w_ref[...] * jnp.tanh(x)

    @pl.when(i == 0)
    def _():
        ssum_ref[...] = jnp.sum(ss_ref[...]).reshape(1, 1)
        viol_ref[...] = jnp.sum(vv_ref[...]).reshape(1, 1)


def _apply_update(nf, w_col, ssum_p, viol_p):
    blk = 1000
    grid = N // blk
    return pl.pallas_call(
        _update_body,
        grid=(grid,),
        in_specs=[
            pl.BlockSpec((blk, D), lambda i: (i, 0)),
            pl.BlockSpec((blk, 1), lambda i: (i, 0)),
            pl.BlockSpec((NW, 16), lambda i: (0, 0)),
            pl.BlockSpec((NW, 16), lambda i: (0, 0)),
        ],
        out_specs=[
            pl.BlockSpec((blk, D), lambda i: (i, 0)),
            pl.BlockSpec((1, 1), lambda i: (0, 0)),
            pl.BlockSpec((1, 1), lambda i: (0, 0)),
        ],
        out_shape=[
            jax.ShapeDtypeStruct((N, D), jnp.float32),
            jax.ShapeDtypeStruct((1, 1), jnp.float32),
            jax.ShapeDtypeStruct((1, 1), jnp.float32),
        ],
    )(nf, w_col, ssum_p, viol_p)


@jax.jit
def _run(node_features, edge_index, W1, b1, W2, b2, W3, b3):
    src = edge_index[0].astype(jnp.int32)
    tgt = edge_index[1].astype(jnp.int32)

    b1d = jnp.concatenate([b1, b1])
    b2d = jnp.concatenate([b2, b2])
    w2d = jnp.zeros((2 * H1, 2 * H2), jnp.float32)
    w2d = w2d.at[:H1, :H2].set(W2).at[H1:, H2:].set(W2)
    w3d = jnp.zeros((2 * H2, 2), jnp.float32)
    w3d = w3d.at[:H2, 0].set(W3[:, 0]).at[H2:, 1].set(W3[:, 0])

    p, q = _project(node_features, W1[:D], W1[D:])
    s = _gather_sum(p, q, src, tgt)
    z = _mlp_tail(s.reshape(E2, 2 * H1), b1d, w2d, b2d, w3d)
    b3x16 = jnp.broadcast_to(b3.reshape(1), (16,))
    scores, wpair, ssum_p, viol_p = _segment_sum(z.reshape(E), src, tgt, b3x16)
    w_col = (wpair[0, :N] + wpair[1, :N]).reshape(N, 1)
    updated, ssum, viol = _apply_update(node_features, w_col, ssum_p, viol_p)

    avg = ssum[0, 0] / jnp.float32(E)
    nviol = viol[0, 0].astype(jnp.int32)
    return updated, scores, avg, nviol


def kernel(node_features, edge_index, node_positions, node_radii,
           W1, b1, W2, b2, W3, b3):
    return _run(node_features, edge_index, W1, b1, W2, b2, W3, b3)


# speedup vs baseline: 13.7516x; 1.3117x over previous
import jax
import jax.numpy as jnp
from jax import lax
from jax.experimental import pallas as pl
from jax.experimental.pallas import tpu as pltpu
from jax.experimental.pallas import tpu_sc as plsc

D = 128
N = 10000
E = 320000
E2 = E // 2
H1 = 64
H2 = 32

NC = 2
NS = 16
NW = NC * NS
EW = E // NW
CHUNK = 80
NCHUNK = EW // CHUNK
NPAD = 10240
SL = NPAD // NS


def _proj_body(nf_ref, w1a_ref, w1b_ref, p_ref, q_ref):
    x = nf_ref[...]
    p_ref[...] = jnp.dot(x, w1a_ref[...], preferred_element_type=jnp.float32)
    q_ref[...] = jnp.dot(x, w1b_ref[...], preferred_element_type=jnp.float32)


def _project(nf, w1a, w1b):
    blk = 1000
    grid = N // blk
    return pl.pallas_call(
        _proj_body,
        grid=(grid,),
        in_specs=[
            pl.BlockSpec((blk, D), lambda i: (i, 0)),
            pl.BlockSpec((D, H1), lambda i: (0, 0)),
            pl.BlockSpec((D, H1), lambda i: (0, 0)),
        ],
        out_specs=[
            pl.BlockSpec((blk, H1), lambda i: (i, 0)),
            pl.BlockSpec((blk, H1), lambda i: (i, 0)),
        ],
        out_shape=[
            jax.ShapeDtypeStruct((N, H1), jnp.float32),
            jax.ShapeDtypeStruct((N, H1), jnp.float32),
        ],
    )(nf, w1a, w1b)


def _gather_sum_body(p_hbm, q_hbm, ei_hbm, out_hbm,
                     idx_s, idx_t, pa, qa, pb, qb,
                     sem_a, sem_b, osem_a, osem_b):
    wid = lax.axis_index("s") * NC + lax.axis_index("c")
    base = wid * EW
    col = base // E2
    r0 = base - col * E2

    pltpu.sync_copy(ei_hbm.at[0, pl.ds(base, EW)], idx_s)
    pltpu.sync_copy(ei_hbm.at[1, pl.ds(base, EW)], idx_t)

    def issue(j, prow, qrow, sem):
        cp = pltpu.async_copy(p_hbm.at[idx_s.at[pl.ds(j * CHUNK, CHUNK)]], prow, sem)
        cq = pltpu.async_copy(q_hbm.at[idx_t.at[pl.ds(j * CHUNK, CHUNK)]], qrow, sem)
        return cp, cq

    def drain(j, prow, qrow, sem, osem):
        pltpu.make_async_copy(p_hbm.at[idx_s.at[pl.ds(j * CHUNK, CHUNK)]], prow, sem).wait()
        pltpu.make_async_copy(q_hbm.at[idx_t.at[pl.ds(j * CHUNK, CHUNK)]], qrow, sem).wait()

        def add_row(r, _):
            for k in range(H1 // 16):
                sl = pl.ds(k * 16, 16)
                plsc.addupdate(prow.at[r, sl], qrow[r, sl])
            return 0

        lax.fori_loop(0, CHUNK, add_row, 0)
        pltpu.async_copy(
            prow, out_hbm.at[pl.ds(r0 + j * CHUNK, CHUNK), pl.ds(col * H1, H1)],
            osem)

    def wait_out(j, prow, osem):
        pltpu.make_async_copy(
            prow, out_hbm.at[pl.ds(r0 + j * CHUNK, CHUNK), pl.ds(col * H1, H1)],
            osem).wait()

    npair = NCHUNK // 2
    issue(0, pa, qa, sem_a)

    def pair_body(g, _):
        j0 = g * 2
        issue(j0 + 1, pb, qb, sem_b)
        drain(j0, pa, qa, sem_a, osem_a)

        @pl.when(j0 + 2 < NCHUNK)
        def _():
            wait_out(j0, pa, osem_a)
            issue(j0 + 2, pa, qa, sem_a)

        drain(j0 + 1, pb, qb, sem_b, osem_b)

        @pl.when(g + 1 < npair)
        def _():
            wait_out(j0 + 1, pb, osem_b)
        return 0

    lax.fori_loop(0, npair, pair_body, 0)
    drain(NCHUNK - 1, pa, qa, sem_a, osem_a)
    wait_out(NCHUNK - 2, pb, osem_b)
    wait_out(NCHUNK - 1, pa, osem_a)


def _gather_sum(p, q, edge_index):
    mesh = plsc.VectorSubcoreMesh(core_axis_name="c", subcore_axis_name="s")
    kern = pl.kernel(
        _gather_sum_body,
        out_type=jax.ShapeDtypeStruct((E2, 2 * H1), jnp.float32),
        mesh=mesh,
        scratch_types=[
            pltpu.VMEM((EW,), jnp.int32),
            pltpu.VMEM((EW,), jnp.int32),
            pltpu.VMEM((CHUNK, H1), jnp.float32),
            pltpu.VMEM((CHUNK, H1), jnp.float32),
            pltpu.VMEM((CHUNK, H1), jnp.float32),
            pltpu.VMEM((CHUNK, H1), jnp.float32),
            pltpu.SemaphoreType.DMA,
            pltpu.SemaphoreType.DMA,
            pltpu.SemaphoreType.DMA,
            pltpu.SemaphoreType.DMA,
        ],
        compiler_params=pltpu.CompilerParams(use_tc_tiling_on_sc=False, needs_layout_passes=False),
    )
    return kern(p, q, edge_index)


def _tail_body(s_ref, b1_ref, w2_ref, b2_ref, w3_ref, b3_ref,
               cs_ref, ssum_ref, viol_ref):
    i = pl.program_id(0)
    x = s_ref[...]
    b1r = b1_ref[...]
    b2r = b2_ref[...]
    w2 = w2_ref[...]
    w3 = w3_ref[...]
    b3 = b3_ref[0, 0]

    ha = jnp.maximum(x[:, :H1] + b1r, 0.0)
    hb = jnp.maximum(x[:, H1:] + b1r, 0.0)
    h2a = jnp.maximum(jnp.dot(ha, w2, preferred_element_type=jnp.float32) + b2r, 0.0)
    h2b = jnp.maximum(jnp.dot(hb, w2, preferred_element_type=jnp.float32) + b2r, 0.0)
    za = jnp.dot(h2a, w3, preferred_element_type=jnp.float32)
    zb = jnp.dot(h2b, w3, preferred_element_type=jnp.float32)
    zT = jnp.concatenate([jnp.transpose(za), jnp.transpose(zb)], axis=0) + b3
    scT = 1.0 / (1.0 + jnp.exp(-zT))
    mT = scT < 0.7
    coeffT = jnp.where(mT, 0.05 * (1.0 - scT), 0.0)
    cs_ref[...] = jnp.concatenate([coeffT, scT], axis=0)

    @pl.when(i == 0)
    def _():
        ssum_ref[...] = jnp.zeros((1, 1), jnp.float32)
        viol_ref[...] = jnp.zeros((1, 1), jnp.float32)

    ssum_ref[...] += jnp.sum(scT).reshape(1, 1)
    viol_ref[...] += jnp.sum(mT.astype(jnp.float32)).reshape(1, 1)


def _mlp_tail(s2, b1, w2, b2, w3, b3):
    blk = 3200
    grid = E2 // blk
    return pl.pallas_call(
        _tail_body,
        grid=(grid,),
        in_specs=[
            pl.BlockSpec((blk, 2 * H1), lambda i: (i, 0)),
            pl.BlockSpec((1, H1), lambda i: (0, 0)),
            pl.BlockSpec((H1, H2), lambda i: (0, 0)),
            pl.BlockSpec((1, H2), lambda i: (0, 0)),
            pl.BlockSpec((H2, 1), lambda i: (0, 0)),
            pl.BlockSpec((1, 1), lambda i: (0, 0)),
        ],
        out_specs=[
            pl.BlockSpec((4, blk), lambda i: (0, i)),
            pl.BlockSpec((1, 1), lambda i: (0, 0)),
            pl.BlockSpec((1, 1), lambda i: (0, 0)),
        ],
        out_shape=[
            jax.ShapeDtypeStruct((4, E2), jnp.float32),
            jax.ShapeDtypeStruct((1, 1), jnp.float32),
            jax.ShapeDtypeStruct((1, 1), jnp.float32),
        ],
    )(s2, b1.reshape(1, H1), w2, b2.reshape(1, H2), w3, b3.reshape(1, 1))


def _segsum_body(cs_hbm, ei_hbm, w_hbm,
                 coeff_v, src_v, tgt_v, w_loc, acc_v, tmp_v, shared):
    cid = lax.axis_index("c")
    sid = lax.axis_index("s")
    wid = sid * NC + cid
    base = wid * EW
    col = base // E2
    r0 = base - col * E2

    pltpu.sync_copy(cs_hbm.at[col, pl.ds(r0, EW)], coeff_v)
    pltpu.sync_copy(ei_hbm.at[0, pl.ds(base, EW)], src_v)
    pltpu.sync_copy(ei_hbm.at[1, pl.ds(base, EW)], tgt_v)

    def zero_body(i, _):
        w_loc[pl.ds(i * 16, 16)] = jnp.zeros((16,), jnp.float32)
        return 0

    lax.fori_loop(0, NPAD // 16, zero_body, 0)

    def edge_body(i, _):
        sl = pl.ds(i * 16, 16)
        cf = coeff_v[sl]
        plsc.addupdate_scatter(w_loc, [src_v[sl]], cf)
        plsc.addupdate_scatter(w_loc, [tgt_v[sl]], cf)
        return 0

    lax.fori_loop(0, EW // 16, edge_body, 0)

    pltpu.sync_copy(w_loc, shared.at[sid])
    plsc.subcore_barrier()

    colsl = pl.ds(sid * SL, SL)
    pltpu.sync_copy(shared.at[0, colsl], acc_v)
    for r in range(1, NS):
        pltpu.sync_copy(shared.at[r, colsl], tmp_v)

        def add_body(i, _):
            sl = pl.ds(i * 16, 16)
            acc_v[sl] += tmp_v[sl]
            return 0

        lax.fori_loop(0, SL // 16, add_body, 0)

    pltpu.sync_copy(acc_v, w_hbm.at[cid, colsl])


def _segment_sum(cs, edge_index):
    mesh = plsc.VectorSubcoreMesh(core_axis_name="c", subcore_axis_name="s")
    kern = pl.kernel(
        _segsum_body,
        out_type=jax.ShapeDtypeStruct((NC, NPAD), jnp.float32),
        mesh=mesh,
        scratch_types=[
            pltpu.VMEM((EW,), jnp.float32),
            pltpu.VMEM((EW,), jnp.int32),
            pltpu.VMEM((EW,), jnp.int32),
            pltpu.VMEM((NPAD,), jnp.float32),
            pltpu.VMEM((SL,), jnp.float32),
            pltpu.VMEM((SL,), jnp.float32),
            pltpu.VMEM_SHARED((NS, NPAD), jnp.float32),
        ],
        compiler_params=pltpu.CompilerParams(use_tc_tiling_on_sc=False, needs_layout_passes=False),
    )
    return kern(cs, edge_index)


def _update_body(nf_ref, w_ref, out_ref):
    x = nf_ref[...]
    out_ref[...] = x + w_ref[...] * jnp.tanh(x)


def _apply_update(nf, w_col):
    blk = 1000
    grid = N // blk
    return pl.pallas_call(
        _update_body,
        grid=(grid,),
        in_specs=[
            pl.BlockSpec((blk, D), lambda i: (i, 0)),
            pl.BlockSpec((blk, 1), lambda i: (i, 0)),
        ],
        out_specs=pl.BlockSpec((blk, D), lambda i: (i, 0)),
        out_shape=jax.ShapeDtypeStruct((N, D), jnp.float32),
    )(nf, w_col)


@jax.jit
def _run(node_features, edge_index, W1, b1, W2, b2, W3, b3):
    ei = edge_index.astype(jnp.int32)

    p, q = _project(node_features, W1[:D], W1[D:])
    s2 = _gather_sum(p, q, ei)
    cs, ssum, viol = _mlp_tail(s2, b1, W2, b2, W3, b3)
    scores = cs[2:4].reshape(E)
    wpair = _segment_sum(cs, ei)
    w_col = (wpair[0, :N] + wpair[1, :N]).reshape(N, 1)
    updated = _apply_update(node_features, w_col)

    avg = ssum[0, 0] / jnp.float32(E)
    nviol = viol[0, 0].astype(jnp.int32)
    return updated, scores, avg, nviol


def kernel(node_features, edge_index, node_positions, node_radii,
           W1, b1, W2, b2, W3, b3):
    return _run(node_features, edge_index, W1, b1, W2, b2, W3, b3)
